# baseline (speedup 1.0000x reference)
import functools

import jax
import jax.numpy as jnp
from jax import lax
from jax.experimental import pallas as pl
from jax.experimental.pallas import tpu as pltpu
from jax.experimental.pallas import tpu_sc as plsc

BATCH = 1024
EMBED = 64
NUM_CORES = 2
NUM_SUBCORES = 16
NUM_WORKERS = NUM_CORES * NUM_SUBCORES
ROWS_PER_WORKER = BATCH // NUM_WORKERS


def _sc_gather(users, items, Gu, Gi):
    mesh = plsc.VectorSubcoreMesh(
        core_axis_name="c", subcore_axis_name="s",
        num_cores=NUM_CORES, num_subcores=NUM_SUBCORES)

    @functools.partial(
        pl.kernel,
        mesh=mesh,
        out_type=(
            jax.ShapeDtypeStruct((BATCH, EMBED), jnp.float32),
            jax.ShapeDtypeStruct((BATCH, EMBED), jnp.float32),
        ),
        scratch_types=(
            pltpu.VMEM((ROWS_PER_WORKER,), jnp.int32),
            pltpu.VMEM((ROWS_PER_WORKER,), jnp.int32),
            pltpu.VMEM((ROWS_PER_WORKER, EMBED), jnp.float32),
            pltpu.VMEM((ROWS_PER_WORKER, EMBED), jnp.float32),
            pltpu.SemaphoreType.DMA,
            pltpu.SemaphoreType.DMA,
        ),
    )
    def gather_kernel(users_hbm, items_hbm, gu_hbm, gi_hbm,
                      gu_out, gi_out, uidx_v, iidx_v, urows_v, irows_v,
                      usem, isem):
        wid = lax.axis_index("s") * NUM_CORES + lax.axis_index("c")
        base = wid * ROWS_PER_WORKER
        pltpu.sync_copy(users_hbm.at[pl.ds(base, ROWS_PER_WORKER)], uidx_v)
        pltpu.sync_copy(items_hbm.at[pl.ds(base, ROWS_PER_WORKER)], iidx_v)
        ucopies = []
        icopies = []
        for r in range(ROWS_PER_WORKER):
            if r % 16 == 0:
                uvec = uidx_v[pl.ds(r, 16)]
                ivec = iidx_v[pl.ds(r, 16)]
            ucopies.append(
                pltpu.async_copy(gu_hbm.at[uvec[r % 16]], urows_v.at[r], usem))
            icopies.append(
                pltpu.async_copy(gi_hbm.at[ivec[r % 16]], irows_v.at[r], isem))
        for c in ucopies:
            c.wait()
        for c in icopies:
            c.wait()
        pltpu.sync_copy(urows_v, gu_out.at[pl.ds(base, ROWS_PER_WORKER)])
        pltpu.sync_copy(irows_v, gi_out.at[pl.ds(base, ROWS_PER_WORKER)])

    return gather_kernel(users, items, Gu, Gi)


def _combine_body(gu_ref, gi_ref, xui_ref):
    colsum = jnp.sum(gi_ref[...], axis=0, keepdims=True)
    xui_ref[...] = gu_ref[...] * colsum


def kernel(users, items, Gu, Gi):
    gamma_u, gamma_i = _sc_gather(users, items, Gu, Gi)
    xui = pl.pallas_call(
        _combine_body,
        out_shape=jax.ShapeDtypeStruct((BATCH, EMBED), jnp.float32),
    )(gamma_u, gamma_i)
    return (xui, gamma_u.reshape(BATCH, 1, EMBED), gamma_i)


# speedup vs baseline: 7.2009x; 7.2009x over previous
import functools

import jax
import jax.numpy as jnp
from jax import lax
from jax.experimental import pallas as pl
from jax.experimental.pallas import tpu as pltpu
from jax.experimental.pallas import tpu_sc as plsc

BATCH = 1024
EMBED = 64
NUM_CORES = 2
NUM_SUBCORES = 16
NUM_WORKERS = NUM_CORES * NUM_SUBCORES
ROWS_PER_WORKER = BATCH // NUM_WORKERS
LANES = 16
NBUF = 4


def _sc_gather(users, items, GuT, GiT):
    mesh = plsc.VectorSubcoreMesh(
        core_axis_name="c", subcore_axis_name="s",
        num_cores=NUM_CORES, num_subcores=NUM_SUBCORES)

    @functools.partial(
        pl.kernel,
        mesh=mesh,
        compiler_params=pltpu.CompilerParams(needs_layout_passes=False),
        out_type=(
            jax.ShapeDtypeStruct((BATCH, EMBED), jnp.float32),
            jax.ShapeDtypeStruct((BATCH, EMBED), jnp.float32),
        ),
        scratch_types=(
            pltpu.VMEM((ROWS_PER_WORKER,), jnp.int32),
            pltpu.VMEM((ROWS_PER_WORKER,), jnp.int32),
            pltpu.VMEM((NBUF, EMBED, 128), jnp.float32),
            pltpu.VMEM((NBUF, EMBED, 128), jnp.float32),
            pltpu.VMEM((ROWS_PER_WORKER, EMBED), jnp.float32),
            pltpu.VMEM((ROWS_PER_WORKER, EMBED), jnp.float32),
            pltpu.SemaphoreType.DMA,
            pltpu.SemaphoreType.DMA,
        ),
    )
    def gather_kernel(users_hbm, items_hbm, gu_hbm, gi_hbm,
                      gu_out, gi_out, uidx_v, iidx_v, ublk_v, iblk_v,
                      urows_v, irows_v, usem, isem):
        wid = lax.axis_index("s") * NUM_CORES + lax.axis_index("c")
        base = wid * ROWS_PER_WORKER
        pltpu.sync_copy(users_hbm.at[pl.ds(base, ROWS_PER_WORKER)], uidx_v)
        pltpu.sync_copy(items_hbm.at[pl.ds(base, ROWS_PER_WORKER)], iidx_v)

        def idx_at(idx_ref, r):
            vec = idx_ref[pl.ds((r // LANES) * LANES, LANES)]
            return vec[r % LANES]

        def fire(table_hbm, idx_ref, blk_ref, sem, r):
            i = idx_at(idx_ref, r)
            blk = pl.multiple_of(lax.shift_right_logical(i, 7) * 128, 128)
            return pltpu.async_copy(
                table_hbm.at[:, pl.ds(blk, 128)], blk_ref.at[r % NBUF], sem)

        def select(idx_ref, blk_ref, rows_ref, r):
            lane = jnp.full((LANES,), jnp.bitwise_and(idx_at(idx_ref, r), 127),
                            jnp.int32)
            for k in range(EMBED // LANES):
                rid = lax.iota(jnp.int32, LANES) + k * LANES
                vals = plsc.load_gather(blk_ref.at[r % NBUF], [rid, lane])
                rows_ref[r, pl.ds(k * LANES, LANES)] = vals

        ucopies = {}
        icopies = {}
        for r in range(NBUF):
            ucopies[r] = fire(gu_hbm, uidx_v, ublk_v, usem, r)
            icopies[r] = fire(gi_hbm, iidx_v, iblk_v, isem, r)
        for r in range(ROWS_PER_WORKER):
            ucopies[r].wait()
            select(uidx_v, ublk_v, urows_v, r)
            if r + NBUF < ROWS_PER_WORKER:
                ucopies[r + NBUF] = fire(gu_hbm, uidx_v, ublk_v, usem, r + NBUF)
            icopies[r].wait()
            select(iidx_v, iblk_v, irows_v, r)
            if r + NBUF < ROWS_PER_WORKER:
                icopies[r + NBUF] = fire(gi_hbm, iidx_v, iblk_v, isem, r + NBUF)
        pltpu.sync_copy(urows_v, gu_out.at[pl.ds(base, ROWS_PER_WORKER)])
        pltpu.sync_copy(irows_v, gi_out.at[pl.ds(base, ROWS_PER_WORKER)])

    return gather_kernel(users, items, GuT, GiT)


def _combine_body(gu_ref, gi_ref, xui_ref):
    colsum = jnp.sum(gi_ref[...], axis=0, keepdims=True)
    xui_ref[...] = gu_ref[...] * colsum


def kernel(users, items, Gu, Gi):
    gamma_u, gamma_i = _sc_gather(users, items, Gu.T, Gi.T)
    xui = pl.pallas_call(
        _combine_body,
        out_shape=jax.ShapeDtypeStruct((BATCH, EMBED), jnp.float32),
    )(gamma_u, gamma_i)
    return (xui, gamma_u.reshape(BATCH, 1, EMBED), gamma_i)


# speedup vs baseline: 7.5643x; 1.0505x over previous
import functools

import jax
import jax.numpy as jnp
from jax import lax
from jax.experimental import pallas as pl
from jax.experimental.pallas import tpu as pltpu
from jax.experimental.pallas import tpu_sc as plsc

BATCH = 1024
EMBED = 64
NUM_CORES = 2
NUM_SUBCORES = 16
NUM_WORKERS = NUM_CORES * NUM_SUBCORES
ROWS_PER_WORKER = BATCH // NUM_WORKERS
LANES = 16
NBUF = 4


def _sc_gather(users, items, GuT, GiT):
    mesh = plsc.VectorSubcoreMesh(
        core_axis_name="c", subcore_axis_name="s",
        num_cores=NUM_CORES, num_subcores=NUM_SUBCORES)

    @functools.partial(
        pl.kernel,
        mesh=mesh,
        compiler_params=pltpu.CompilerParams(needs_layout_passes=False),
        out_type=(
            jax.ShapeDtypeStruct((BATCH, EMBED), jnp.float32),
            jax.ShapeDtypeStruct((BATCH, EMBED), jnp.float32),
        ),
        scratch_types=(
            pltpu.VMEM((ROWS_PER_WORKER,), jnp.int32),
            pltpu.VMEM((ROWS_PER_WORKER,), jnp.int32),
            pltpu.VMEM((NBUF, EMBED, 128), jnp.float32),
            pltpu.VMEM((NBUF, EMBED, 128), jnp.float32),
            pltpu.VMEM((ROWS_PER_WORKER, EMBED), jnp.float32),
            pltpu.VMEM((ROWS_PER_WORKER, EMBED), jnp.float32),
            pltpu.SMEM((ROWS_PER_WORKER,), jnp.int32),
            pltpu.SMEM((ROWS_PER_WORKER,), jnp.int32),
            pltpu.SMEM((ROWS_PER_WORKER,), jnp.int32),
            pltpu.SMEM((ROWS_PER_WORKER,), jnp.int32),
            pltpu.SemaphoreType.DMA,
            pltpu.SemaphoreType.DMA,
        ),
    )
    def gather_kernel(users_hbm, items_hbm, gu_hbm, gi_hbm,
                      gu_out, gi_out, uidx_v, iidx_v, ublk_v, iblk_v,
                      urows_v, irows_v, ublk_s, ulane_s, iblk_s, ilane_s,
                      usem, isem):
        wid = lax.axis_index("s") * NUM_CORES + lax.axis_index("c")
        base = wid * ROWS_PER_WORKER
        pltpu.sync_copy(users_hbm.at[pl.ds(base, ROWS_PER_WORKER)], uidx_v)
        pltpu.sync_copy(items_hbm.at[pl.ds(base, ROWS_PER_WORKER)], iidx_v)

        for c in range(ROWS_PER_WORKER // LANES):
            uvec = uidx_v[pl.ds(c * LANES, LANES)]
            ivec = iidx_v[pl.ds(c * LANES, LANES)]
            ublkv = jnp.bitwise_and(uvec, ~127)
            ulanev = jnp.bitwise_and(uvec, 127)
            iblkv = jnp.bitwise_and(ivec, ~127)
            ilanev = jnp.bitwise_and(ivec, 127)
            for k in range(LANES):
                r = c * LANES + k
                ublk_s[r] = ublkv[k]
                ulane_s[r] = ulanev[k]
                iblk_s[r] = iblkv[k]
                ilane_s[r] = ilanev[k]

        def fire(table_hbm, blk_s, blk_ref, sem, r, slot):
            blk = pl.multiple_of(blk_s[r], 128)
            pltpu.async_copy(
                table_hbm.at[:, pl.ds(blk, 128)], blk_ref.at[slot], sem)

        def drain(table_hbm, blk_ref, sem):
            pltpu.make_async_copy(
                table_hbm.at[:, pl.ds(0, 128)], blk_ref.at[0], sem).wait()

        def select(lane_s, blk_ref, rows_ref, r, slot):
            lane = jnp.full((LANES,), lane_s[r], jnp.int32)
            row = jnp.full((LANES,), r, jnp.int32)
            for k in range(EMBED // LANES):
                rid = lax.iota(jnp.int32, LANES) + k * LANES
                vals = plsc.load_gather(blk_ref.at[slot], [rid, lane])
                plsc.store_scatter(rows_ref, [row, rid], vals)

        for r in range(NBUF):
            fire(gu_hbm, ublk_s, ublk_v, usem, r, r)
            fire(gi_hbm, iblk_s, iblk_v, isem, r, r)

        def body(r, carry):
            slot = lax.rem(r, NBUF)
            drain(gu_hbm, ublk_v, usem)
            select(ulane_s, ublk_v, urows_v, r, slot)
            drain(gi_hbm, iblk_v, isem)
            select(ilane_s, iblk_v, irows_v, r, slot)

            @pl.when(r < ROWS_PER_WORKER - NBUF)
            def _():
                fire(gu_hbm, ublk_s, ublk_v, usem, r + NBUF, slot)
                fire(gi_hbm, iblk_s, iblk_v, isem, r + NBUF, slot)
            return carry

        lax.fori_loop(0, ROWS_PER_WORKER, body, 0)
        pltpu.sync_copy(urows_v, gu_out.at[pl.ds(base, ROWS_PER_WORKER)])
        pltpu.sync_copy(irows_v, gi_out.at[pl.ds(base, ROWS_PER_WORKER)])

    return gather_kernel(users, items, GuT, GiT)


def _combine_body(gu_ref, gi_ref, xui_ref):
    colsum = jnp.sum(gi_ref[...], axis=0, keepdims=True)
    xui_ref[...] = gu_ref[...] * colsum


def kernel(users, items, Gu, Gi):
    gamma_u, gamma_i = _sc_gather(users, items, Gu.T, Gi.T)
    xui = pl.pallas_call(
        _combine_body,
        out_shape=jax.ShapeDtypeStruct((BATCH, EMBED), jnp.float32),
    )(gamma_u, gamma_i)
    return (xui, gamma_u.reshape(BATCH, 1, EMBED), gamma_i)


# speedup vs baseline: 8.3985x; 1.1103x over previous
import functools

import jax
import jax.numpy as jnp
from jax import lax
from jax.experimental import pallas as pl
from jax.experimental.pallas import tpu as pltpu
from jax.experimental.pallas import tpu_sc as plsc

BATCH = 1024
EMBED = 64
NUM_CORES = 2
NUM_SUBCORES = 16
NUM_WORKERS = NUM_CORES * NUM_SUBCORES
ROWS_PER_WORKER = BATCH // NUM_WORKERS
LANES = 16
NBUF = 6


def _sc_gather(users, items, GuT, GiT):
    mesh = plsc.VectorSubcoreMesh(
        core_axis_name="c", subcore_axis_name="s",
        num_cores=NUM_CORES, num_subcores=NUM_SUBCORES)

    @functools.partial(
        pl.kernel,
        mesh=mesh,
        compiler_params=pltpu.CompilerParams(needs_layout_passes=False),
        out_type=(
            jax.ShapeDtypeStruct((BATCH, EMBED), jnp.float32),
            jax.ShapeDtypeStruct((BATCH, EMBED), jnp.float32),
        ),
        scratch_types=(
            pltpu.VMEM((ROWS_PER_WORKER,), jnp.int32),
            pltpu.VMEM((ROWS_PER_WORKER,), jnp.int32),
            pltpu.VMEM((NBUF, EMBED, 128), jnp.float32),
            pltpu.VMEM((NBUF, EMBED, 128), jnp.float32),
            pltpu.VMEM((ROWS_PER_WORKER, EMBED), jnp.float32),
            pltpu.VMEM((ROWS_PER_WORKER, EMBED), jnp.float32),
            pltpu.SMEM((ROWS_PER_WORKER,), jnp.int32),
            pltpu.SMEM((ROWS_PER_WORKER,), jnp.int32),
            pltpu.SMEM((ROWS_PER_WORKER,), jnp.int32),
            pltpu.SMEM((ROWS_PER_WORKER,), jnp.int32),
            pltpu.SemaphoreType.DMA,
            pltpu.SemaphoreType.DMA,
        ),
    )
    def gather_kernel(users_hbm, items_hbm, gu_hbm, gi_hbm,
                      gu_out, gi_out, uidx_v, iidx_v, ublk_v, iblk_v,
                      urows_v, irows_v, ublk_s, ulane_s, iblk_s, ilane_s,
                      usem, isem):
        wid = lax.axis_index("s") * NUM_CORES + lax.axis_index("c")
        base = wid * ROWS_PER_WORKER
        pltpu.sync_copy(users_hbm.at[pl.ds(base, ROWS_PER_WORKER)], uidx_v)
        pltpu.sync_copy(items_hbm.at[pl.ds(base, ROWS_PER_WORKER)], iidx_v)

        for c in range(ROWS_PER_WORKER // LANES):
            uvec = uidx_v[pl.ds(c * LANES, LANES)]
            ivec = iidx_v[pl.ds(c * LANES, LANES)]
            ublkv = jnp.bitwise_and(uvec, ~127)
            ulanev = jnp.bitwise_and(uvec, 127)
            iblkv = jnp.bitwise_and(ivec, ~127)
            ilanev = jnp.bitwise_and(ivec, 127)
            for k in range(LANES):
                r = c * LANES + k
                ublk_s[r] = ublkv[k]
                ulane_s[r] = ulanev[k]
                iblk_s[r] = iblkv[k]
                ilane_s[r] = ilanev[k]

        def fire(table_hbm, blk_s, blk_ref, sem, r, slot):
            blk = pl.multiple_of(blk_s[r], 128)
            pltpu.async_copy(
                table_hbm.at[:, pl.ds(blk, 128)], blk_ref.at[slot], sem)

        def drain(table_hbm, blk_ref, sem):
            pltpu.make_async_copy(
                table_hbm.at[:, pl.ds(0, 128)], blk_ref.at[0], sem).wait()

        def select(lane_s, blk_ref, rows_ref, r, slot):
            lane = jnp.full((LANES,), lane_s[r], jnp.int32)
            row = jnp.full((LANES,), r, jnp.int32)
            for k in range(EMBED // LANES):
                rid = lax.iota(jnp.int32, LANES) + k * LANES
                vals = plsc.load_gather(blk_ref.at[slot], [rid, lane])
                plsc.store_scatter(rows_ref, [row, rid], vals)

        for r in range(NBUF):
            fire(gu_hbm, ublk_s, ublk_v, usem, r, r)
            fire(gi_hbm, iblk_s, iblk_v, isem, r, r)

        def body(r, carry):
            slot = lax.rem(r, NBUF)
            drain(gu_hbm, ublk_v, usem)
            select(ulane_s, ublk_v, urows_v, r, slot)
            drain(gi_hbm, iblk_v, isem)
            select(ilane_s, iblk_v, irows_v, r, slot)

            @pl.when(r < ROWS_PER_WORKER - NBUF)
            def _():
                fire(gu_hbm, ublk_s, ublk_v, usem, r + NBUF, slot)
                fire(gi_hbm, iblk_s, iblk_v, isem, r + NBUF, slot)
            return carry

        lax.fori_loop(0, ROWS_PER_WORKER, body, 0)
        pltpu.sync_copy(urows_v, gu_out.at[pl.ds(base, ROWS_PER_WORKER)])
        pltpu.sync_copy(irows_v, gi_out.at[pl.ds(base, ROWS_PER_WORKER)])

    return gather_kernel(users, items, GuT, GiT)


def _combine_body(gu_ref, gi_ref, guT_ref, giT_ref, xuiT_ref):
    gu = gu_ref[...]
    gi = gi_ref[...]
    colsum = jnp.sum(gi, axis=0, keepdims=True)
    guT = gu.T
    guT_ref[...] = guT
    giT_ref[...] = gi.T
    xuiT_ref[...] = guT * colsum.T


def kernel(users, items, Gu, Gi):
    gamma_u, gamma_i = _sc_gather(users, items, Gu.T, Gi.T)
    guT, giT, xuiT = pl.pallas_call(
        _combine_body,
        out_shape=(
            jax.ShapeDtypeStruct((EMBED, BATCH), jnp.float32),
            jax.ShapeDtypeStruct((EMBED, BATCH), jnp.float32),
            jax.ShapeDtypeStruct((EMBED, BATCH), jnp.float32),
        ),
    )(gamma_u, gamma_i)
    return (xuiT.T, guT.T.reshape(BATCH, 1, EMBED), giT.T)


# speedup vs baseline: 8.4316x; 1.0039x over previous
import functools

import jax
import jax.numpy as jnp
from jax import lax
from jax.experimental import pallas as pl
from jax.experimental.pallas import tpu as pltpu
from jax.experimental.pallas import tpu_sc as plsc

BATCH = 1024
EMBED = 64
NUM_CORES = 2
NUM_SUBCORES = 16
NUM_WORKERS = NUM_CORES * NUM_SUBCORES
ROWS_PER_WORKER = BATCH // NUM_WORKERS
LANES = 16
NBUF = 6


def _sc_gather(users, items, GuT, GiT):
    mesh = plsc.VectorSubcoreMesh(
        core_axis_name="c", subcore_axis_name="s",
        num_cores=NUM_CORES, num_subcores=NUM_SUBCORES)

    @functools.partial(
        pl.kernel,
        mesh=mesh,
        compiler_params=pltpu.CompilerParams(
            needs_layout_passes=False, skip_device_barrier=True),
        out_type=(
            jax.ShapeDtypeStruct((BATCH, EMBED), jnp.float32),
            jax.ShapeDtypeStruct((BATCH, EMBED), jnp.float32),
        ),
        scratch_types=(
            pltpu.VMEM((ROWS_PER_WORKER,), jnp.int32),
            pltpu.VMEM((ROWS_PER_WORKER,), jnp.int32),
            pltpu.VMEM((NBUF, EMBED, 128), jnp.float32),
            pltpu.VMEM((NBUF, EMBED, 128), jnp.float32),
            pltpu.VMEM((ROWS_PER_WORKER, EMBED), jnp.float32),
            pltpu.VMEM((ROWS_PER_WORKER, EMBED), jnp.float32),
            pltpu.SMEM((ROWS_PER_WORKER,), jnp.int32),
            pltpu.SMEM((ROWS_PER_WORKER,), jnp.int32),
            pltpu.SMEM((ROWS_PER_WORKER,), jnp.int32),
            pltpu.SMEM((ROWS_PER_WORKER,), jnp.int32),
            pltpu.SemaphoreType.DMA,
            pltpu.SemaphoreType.DMA,
        ),
    )
    def gather_kernel(users_hbm, items_hbm, gu_hbm, gi_hbm,
                      gu_out, gi_out, uidx_v, iidx_v, ublk_v, iblk_v,
                      urows_v, irows_v, ublk_s, ulane_s, iblk_s, ilane_s,
                      usem, isem):
        wid = lax.axis_index("s") * NUM_CORES + lax.axis_index("c")
        base = wid * ROWS_PER_WORKER
        pltpu.sync_copy(users_hbm.at[pl.ds(base, ROWS_PER_WORKER)], uidx_v)
        pltpu.sync_copy(items_hbm.at[pl.ds(base, ROWS_PER_WORKER)], iidx_v)

        for c in range(ROWS_PER_WORKER // LANES):
            uvec = uidx_v[pl.ds(c * LANES, LANES)]
            ivec = iidx_v[pl.ds(c * LANES, LANES)]
            ublkv = jnp.bitwise_and(uvec, ~127)
            ulanev = jnp.bitwise_and(uvec, 127)
            iblkv = jnp.bitwise_and(ivec, ~127)
            ilanev = jnp.bitwise_and(ivec, 127)
            for k in range(LANES):
                r = c * LANES + k
                ublk_s[r] = ublkv[k]
                ulane_s[r] = ulanev[k]
                iblk_s[r] = iblkv[k]
                ilane_s[r] = ilanev[k]

        def fire(table_hbm, blk_s, blk_ref, sem, r, slot):
            blk = pl.multiple_of(blk_s[r], 128)
            pltpu.async_copy(
                table_hbm.at[:, pl.ds(blk, 128)], blk_ref.at[slot], sem)

        def drain(table_hbm, blk_ref, sem):
            pltpu.make_async_copy(
                table_hbm.at[:, pl.ds(0, 128)], blk_ref.at[0], sem).wait()

        def select(lane_s, blk_ref, rows_ref, r, slot):
            lane = jnp.full((LANES,), lane_s[r], jnp.int32)
            row = jnp.full((LANES,), r, jnp.int32)
            for k in range(EMBED // LANES):
                rid = lax.iota(jnp.int32, LANES) + k * LANES
                vals = plsc.load_gather(blk_ref.at[slot], [rid, lane])
                plsc.store_scatter(rows_ref, [row, rid], vals)

        for r in range(NBUF):
            fire(gu_hbm, ublk_s, ublk_v, usem, r, r)
            fire(gi_hbm, iblk_s, iblk_v, isem, r, r)

        def body(r, carry):
            slot = lax.rem(r, NBUF)
            drain(gu_hbm, ublk_v, usem)
            select(ulane_s, ublk_v, urows_v, r, slot)
            drain(gi_hbm, iblk_v, isem)
            select(ilane_s, iblk_v, irows_v, r, slot)

            @pl.when(r < ROWS_PER_WORKER - NBUF)
            def _():
                fire(gu_hbm, ublk_s, ublk_v, usem, r + NBUF, slot)
                fire(gi_hbm, iblk_s, iblk_v, isem, r + NBUF, slot)
            return carry

        lax.fori_loop(0, ROWS_PER_WORKER, body, 0)
        pltpu.sync_copy(urows_v, gu_out.at[pl.ds(base, ROWS_PER_WORKER)])
        pltpu.sync_copy(irows_v, gi_out.at[pl.ds(base, ROWS_PER_WORKER)])

    return gather_kernel(users, items, GuT, GiT)


def _combine_body(gu_ref, gi_ref, guT_ref, giT_ref, xuiT_ref):
    gu = gu_ref[...]
    gi = gi_ref[...]
    colsum = jnp.sum(gi, axis=0, keepdims=True)
    guT = gu.T
    guT_ref[...] = guT
    giT_ref[...] = gi.T
    xuiT_ref[...] = guT * colsum.T


def kernel(users, items, Gu, Gi):
    gamma_u, gamma_i = _sc_gather(users, items, Gu.T, Gi.T)
    guT, giT, xuiT = pl.pallas_call(
        _combine_body,
        out_shape=(
            jax.ShapeDtypeStruct((EMBED, BATCH), jnp.float32),
            jax.ShapeDtypeStruct((EMBED, BATCH), jnp.float32),
            jax.ShapeDtypeStruct((EMBED, BATCH), jnp.float32),
        ),
    )(gamma_u, gamma_i)
    return (xuiT.T, guT.T.reshape(BATCH, 1, EMBED), giT.T)
